# baseline (speedup 1.0000x reference)
import functools

import jax
import jax.numpy as jnp
from jax import lax
from jax.experimental import pallas as pl
from jax.experimental.pallas import tpu as pltpu
from jax.experimental.pallas import tpu_sc as plsc

_TIME = 288
_WEEK = 8
_F = 64
_NB = 1024
_L = 16
_NC = 2


def _sc_tr_body(T, NBLK, x_hbm, pk_hbm):
    wid = lax.axis_index("s") * _NC + lax.axis_index("c")
    nb = wid & 15
    bh = wid >> 4
    n0 = nb * NBLK
    B = x_hbm.shape[0]
    rows = NBLK * T
    half = B // 2

    GW = 3 * _L // T
    iota = lax.broadcasted_iota(jnp.int32, (_L,), 0)

    def inner(xtile, pktile):
        def per_b(b, _):
            pltpu.sync_copy(x_hbm.at[b, :, pl.ds(n0, NBLK)], xtile)

            def per_chunk(m, _):
                for q in range(3):
                    i = iota + q * _L
                    n_off = (i * 21846) >> 18
                    t = i - n_off * T
                    n = n_off + m * GW
                    v = plsc.load_gather(xtile, [t, n])
                    pktile[pl.ds(m * 3 * _L + q * _L, _L)] = v
                return 0

            lax.fori_loop(0, rows // (3 * _L), per_chunk, 0)
            pltpu.sync_copy(pktile, pk_hbm.at[b, pl.ds(n0 * T, rows)])
            return 0

        lax.fori_loop(bh * half, (bh + 1) * half, per_b, 0)

    pl.run_scoped(
        inner,
        pltpu.VMEM((T, NBLK), jnp.float32),
        pltpu.VMEM((NBLK * T,), jnp.float32),
    )


def _tc_body(pk_ref, tdt_ref, twt_ref, out_ref):
    p = pk_ref[0].astype(jnp.int32)
    didx = p >> 3
    widx = p & 7
    J = p.shape[1]
    kd = lax.broadcasted_iota(jnp.int32, (_TIME, J), 0)
    kw = lax.broadcasted_iota(jnp.int32, (_WEEK, J), 0)
    ohd = (didx == kd).astype(jnp.bfloat16)
    ohw = (widx == kw).astype(jnp.bfloat16)
    acc = lax.dot_general(
        tdt_ref[...], ohd, (((1,), (0,)), ((), ())),
        preferred_element_type=jnp.float32)
    acc += lax.dot_general(
        twt_ref[...], ohw, (((1,), (0,)), ((), ())),
        preferred_element_type=jnp.float32)
    out_ref[0] = acc


@jax.jit
def kernel(x, time_day, time_week):
    B, T, N, C = x.shape
    F = time_day.shape[1]
    scale = jnp.asarray([0.0, float(_TIME), 1.0], jnp.float32)
    weight = jnp.asarray([0.0, 8.0, 1.0], jnp.float32)
    pk_nat = jnp.floor(x * scale) @ weight
    NBLK = N // 16
    sc_tr = functools.partial(
        pl.kernel,
        mesh=plsc.VectorSubcoreMesh(core_axis_name="c", subcore_axis_name="s"),
        out_type=jax.ShapeDtypeStruct((B, N * T), jnp.float32),
        compiler_params=pltpu.CompilerParams(needs_layout_passes=False),
    )(functools.partial(_sc_tr_body, T, NBLK))
    pkT = sc_tr(pk_nat).reshape(B, 1, N * T)
    tdt = time_day.T.astype(jnp.bfloat16)
    twt = jnp.pad(time_week, ((0, _WEEK - time_week.shape[0]), (0, 0)))
    twt = twt.T.astype(jnp.bfloat16)

    J = _NB * T
    out_flat = pl.pallas_call(
        _tc_body,
        grid=(B, N // _NB),
        in_specs=[
            pl.BlockSpec((1, 1, J), lambda b, n: (b, 0, n)),
            pl.BlockSpec((F, _TIME), lambda b, n: (0, 0)),
            pl.BlockSpec((F, _WEEK), lambda b, n: (0, 0)),
        ],
        out_specs=pl.BlockSpec((1, F, J), lambda b, n: (b, 0, n)),
        out_shape=jax.ShapeDtypeStruct((B, F, N * T), jnp.float32),
        compiler_params=pltpu.CompilerParams(
            dimension_semantics=("parallel", "parallel")),
    )(pkT, tdt, twt)
    return out_flat.reshape(B, F, N, T)


# speedup vs baseline: 1.0165x; 1.0165x over previous
import functools

import jax
import jax.numpy as jnp
from jax import lax
from jax.experimental import pallas as pl
from jax.experimental.pallas import tpu as pltpu
from jax.experimental.pallas import tpu_sc as plsc

_TIME = 288
_WEEK = 8
_F = 64
_NB = 2048
_L = 16
_NC = 2


def _sc_tr_body(T, NBLK, x_hbm, pk_hbm):
    wid = lax.axis_index("s") * _NC + lax.axis_index("c")
    nb = wid & 15
    bh = wid >> 4
    n0 = nb * NBLK
    B = x_hbm.shape[0]
    rows = NBLK * T
    half = B // 2

    GW = 3 * _L // T
    iota = lax.broadcasted_iota(jnp.int32, (_L,), 0)

    def inner(xtile, pktile):
        def per_b(b, _):
            pltpu.sync_copy(x_hbm.at[b, :, pl.ds(n0, NBLK)], xtile)

            def per_chunk(m, _):
                for q in range(3):
                    i = iota + q * _L
                    n_off = (i * 21846) >> 18
                    t = i - n_off * T
                    n = n_off + m * GW
                    v = plsc.load_gather(xtile, [t, n])
                    pktile[pl.ds(m * 3 * _L + q * _L, _L)] = v
                return 0

            lax.fori_loop(0, rows // (3 * _L), per_chunk, 0)
            pltpu.sync_copy(pktile, pk_hbm.at[b, pl.ds(n0 * T, rows)])
            return 0

        lax.fori_loop(bh * half, (bh + 1) * half, per_b, 0)

    pl.run_scoped(
        inner,
        pltpu.VMEM((T, NBLK), jnp.float32),
        pltpu.VMEM((NBLK * T,), jnp.float32),
    )


def _tc_body(pk_ref, tdt_ref, twt_ref, out_ref):
    p = pk_ref[0].astype(jnp.int32)
    didx = p >> 3
    widx = p & 7
    J = p.shape[1]
    kd = lax.broadcasted_iota(jnp.int32, (_TIME, J), 0)
    kw = lax.broadcasted_iota(jnp.int32, (_WEEK, J), 0)
    ohd = (didx == kd).astype(jnp.bfloat16)
    ohw = (widx == kw).astype(jnp.bfloat16)
    acc = lax.dot_general(
        tdt_ref[...], ohd, (((1,), (0,)), ((), ())),
        preferred_element_type=jnp.float32)
    acc += lax.dot_general(
        twt_ref[...], ohw, (((1,), (0,)), ((), ())),
        preferred_element_type=jnp.float32)
    out_ref[0] = acc


@jax.jit
def kernel(x, time_day, time_week):
    B, T, N, C = x.shape
    F = time_day.shape[1]
    pk_nat = jnp.floor(x[..., 1] * _TIME) * 8.0 + jnp.floor(x[..., 2])
    NBLK = N // 16
    sc_tr = functools.partial(
        pl.kernel,
        mesh=plsc.VectorSubcoreMesh(core_axis_name="c", subcore_axis_name="s"),
        out_type=jax.ShapeDtypeStruct((B, N * T), jnp.float32),
        compiler_params=pltpu.CompilerParams(needs_layout_passes=False),
    )(functools.partial(_sc_tr_body, T, NBLK))
    pkT = sc_tr(pk_nat).reshape(B, 1, N * T)
    tdt = time_day.T.astype(jnp.bfloat16)
    twt = jnp.pad(time_week, ((0, _WEEK - time_week.shape[0]), (0, 0)))
    twt = twt.T.astype(jnp.bfloat16)

    J = _NB * T
    out_flat = pl.pallas_call(
        _tc_body,
        grid=(B, N // _NB),
        in_specs=[
            pl.BlockSpec((1, 1, J), lambda b, n: (b, 0, n)),
            pl.BlockSpec((F, _TIME), lambda b, n: (0, 0)),
            pl.BlockSpec((F, _WEEK), lambda b, n: (0, 0)),
        ],
        out_specs=pl.BlockSpec((1, F, J), lambda b, n: (b, 0, n)),
        out_shape=jax.ShapeDtypeStruct((B, F, N * T), jnp.float32),
        compiler_params=pltpu.CompilerParams(
            dimension_semantics=("parallel", "parallel")),
    )(pkT, tdt, twt)
    return out_flat.reshape(B, F, N, T)
